# initial kernel scaffold (unmeasured)
import jax
import jax.numpy as jnp
from jax import lax
from jax.experimental import pallas as pl
from jax.experimental.pallas import tpu as pltpu

N_DEV = 4
N_TOK = 2048
D_IN = 512
D_H = 1024
N_EXP = 32
E_LOC = N_EXP // N_DEV
CHUNK = N_TOK // N_DEV
N_HOPS = 2 * (N_DEV - 1)


def kernel(x, router_W, route_idx, expert_W, shared_W):
    def body(x_ref, rw_ref, ri_ref, ew_ref, sw_ref, out_ref,
             sbuf, comm, send_sems, recv_sems):
        my = lax.axis_index("i")
        left = lax.rem(my + N_DEV - 1, N_DEV)
        right = lax.rem(my + 1, N_DEV)

        barrier = pltpu.get_barrier_semaphore()
        for nbr in (left, right):
            pl.semaphore_signal(barrier, inc=1, device_id=(nbr,),
                                device_id_type=pl.DeviceIdType.MESH)
        pl.semaphore_wait(barrier, 2)

        xf = x_ref[:, :]
        scores = jnp.dot(xf, rw_ref[:, :], preferred_element_type=jnp.float32)
        s_max = jnp.max(scores, axis=-1, keepdims=True)
        p = jnp.exp(scores - s_max)
        probs = p / jnp.sum(p, axis=-1, keepdims=True)
        ri = ri_ref[:, :]
        eidx = lax.broadcasted_iota(jnp.int32, (N_TOK, N_EXP), 1)
        p_top = jnp.sum(jnp.where(eidx == ri, probs, 0.0),
                        axis=-1, keepdims=True)

        xb = xf.astype(jnp.bfloat16)
        e_base = my * E_LOC
        acc = jnp.zeros((N_TOK, D_H), jnp.float32)
        for le in range(E_LOC):
            gate = jnp.where(ri == e_base + le, p_top, 0.0)
            xe = xb * gate.astype(jnp.bfloat16)
            acc = acc + jnp.dot(xe, ew_ref[le].astype(jnp.bfloat16),
                                preferred_element_type=jnp.float32)
        out_ref[:, :] = acc

        def rdma(src, k):
            return pltpu.make_async_remote_copy(
                src_ref=src, dst_ref=comm.at[k],
                send_sem=send_sems.at[k], recv_sem=recv_sems.at[k],
                device_id=(right,), device_id_type=pl.DeviceIdType.MESH)

        for s in range(N_DEV - 1):
            cs = lax.rem(my - s + N_DEV, N_DEV)
            part = out_ref[pl.ds(cs * CHUNK, CHUNK), :]
            if s == 0:
                sbuf[:, :] = part.astype(jnp.bfloat16)
            else:
                sbuf[:, :] = (comm[s - 1].astype(jnp.float32)
                              + part).astype(jnp.bfloat16)
            op = rdma(sbuf, s)
            op.start()
            op.wait()

        cmine = lax.rem(my + 1, N_DEV)
        xc = x_ref[pl.ds(cmine * CHUNK, CHUNK), :].astype(jnp.bfloat16)
        shared_c = jnp.dot(xc, sw_ref[:, :].astype(jnp.bfloat16),
                           preferred_element_type=jnp.float32)
        final = (comm[N_DEV - 2].astype(jnp.float32)
                 + out_ref[pl.ds(cmine * CHUNK, CHUNK), :]
                 + shared_c)
        out_ref[pl.ds(cmine * CHUNK, CHUNK), :] = final

        sbuf[:, :] = final.astype(jnp.bfloat16)
        for t in range(N_DEV - 1):
            k = (N_DEV - 1) + t
            op = rdma(sbuf if t == 0 else comm.at[k - 1], k)
            op.start()
            op.wait()
            corig = lax.rem(my - t + N_DEV, N_DEV)
            out_ref[pl.ds(corig * CHUNK, CHUNK), :] = comm[k].astype(jnp.float32)

    return pl.pallas_call(
        body,
        out_shape=jax.ShapeDtypeStruct((N_TOK, D_H), jnp.float32),
        in_specs=[pl.BlockSpec(memory_space=pltpu.VMEM)] * 5,
        out_specs=pl.BlockSpec(memory_space=pltpu.VMEM),
        scratch_shapes=[
            pltpu.VMEM((CHUNK, D_H), jnp.bfloat16),
            pltpu.VMEM((N_HOPS, CHUNK, D_H), jnp.bfloat16),
            pltpu.SemaphoreType.DMA((N_HOPS,)),
            pltpu.SemaphoreType.DMA((N_HOPS,)),
        ],
        compiler_params=pltpu.CompilerParams(collective_id=0),
    )(x, router_W, route_idx, expert_W, shared_W)


# baseline (device time: 123935 ns/iter reference)
import jax
import jax.numpy as jnp
from jax import lax
from jax.experimental import pallas as pl
from jax.experimental.pallas import tpu as pltpu

N_DEV = 4
N_TOK = 2048
D_IN = 512
D_H = 1024
N_EXP = 32
E_LOC = N_EXP // N_DEV
CHUNK = N_TOK // N_DEV
N_HOPS = 2 * (N_DEV - 1)


def kernel(x, router_W, route_idx, expert_W, shared_W):
    def body(x_ref, rw_ref, ri_ref, ew_ref, sw_ref, out_ref,
             pt_scr, sbuf, comm, send_sems, recv_sems):
        my = lax.axis_index("i")
        left = lax.rem(my + N_DEV - 1, N_DEV)
        right = lax.rem(my + 1, N_DEV)

        barrier = pltpu.get_barrier_semaphore()
        for nbr in (left, right):
            pl.semaphore_signal(barrier, inc=1, device_id=(nbr,),
                                device_id_type=pl.DeviceIdType.MESH)
        pl.semaphore_wait(barrier, 2)

        scores = jnp.dot(x_ref[:, :], rw_ref[:, :],
                         preferred_element_type=jnp.float32)
        s_max = jnp.max(scores, axis=-1, keepdims=True)
        p = jnp.exp(scores - s_max)
        probs = p / jnp.sum(p, axis=-1, keepdims=True)
        ri = ri_ref[:, :]
        eidx = lax.broadcasted_iota(jnp.int32, (N_TOK, N_EXP), 1)
        pt_scr[:, :] = jnp.sum(jnp.where(eidx == ri, probs, 0.0),
                               axis=-1, keepdims=True)

        e_base = my * E_LOC
        for c in range(N_DEV):
            r0 = c * CHUNK
            xb = x_ref[pl.ds(r0, CHUNK), :].astype(jnp.bfloat16)
            ri_c = ri_ref[pl.ds(r0, CHUNK), :]
            pt_c = pt_scr[pl.ds(r0, CHUNK), :]
            acc = jnp.zeros((CHUNK, D_H), jnp.float32)
            for le in range(E_LOC):
                gate = jnp.where(ri_c == e_base + le, pt_c, 0.0)
                acc = acc + jnp.dot(xb * gate.astype(jnp.bfloat16), ew_ref[le],
                                    preferred_element_type=jnp.float32)
            out_ref[pl.ds(r0, CHUNK), :] = acc

        def rdma(src, k):
            return pltpu.make_async_remote_copy(
                src_ref=src, dst_ref=comm.at[k],
                send_sem=send_sems.at[k], recv_sem=recv_sems.at[k],
                device_id=(right,), device_id_type=pl.DeviceIdType.MESH)

        for s in range(N_DEV - 1):
            cs = lax.rem(my - s + N_DEV, N_DEV)
            part = out_ref[pl.ds(cs * CHUNK, CHUNK), :]
            if s == 0:
                sbuf[:, :] = part.astype(jnp.bfloat16)
            else:
                sbuf[:, :] = (comm[s - 1].astype(jnp.float32)
                              + part).astype(jnp.bfloat16)
            op = rdma(sbuf, s)
            op.start()
            op.wait()

        cmine = lax.rem(my + 1, N_DEV)
        xc = x_ref[pl.ds(cmine * CHUNK, CHUNK), :].astype(jnp.bfloat16)
        shared_c = jnp.dot(xc, sw_ref[:, :],
                           preferred_element_type=jnp.float32)
        final = (comm[N_DEV - 2].astype(jnp.float32)
                 + out_ref[pl.ds(cmine * CHUNK, CHUNK), :]
                 + shared_c)
        out_ref[pl.ds(cmine * CHUNK, CHUNK), :] = final

        sbuf[:, :] = final.astype(jnp.bfloat16)
        for t in range(N_DEV - 1):
            k = (N_DEV - 1) + t
            op = rdma(sbuf if t == 0 else comm.at[k - 1], k)
            op.start()
            op.wait()
            corig = lax.rem(my - t + N_DEV, N_DEV)
            out_ref[pl.ds(corig * CHUNK, CHUNK), :] = comm[k].astype(jnp.float32)

    return pl.pallas_call(
        body,
        out_shape=jax.ShapeDtypeStruct((N_TOK, D_H), jnp.float32),
        in_specs=[pl.BlockSpec(memory_space=pltpu.VMEM)] * 5,
        out_specs=pl.BlockSpec(memory_space=pltpu.VMEM),
        scratch_shapes=[
            pltpu.VMEM((N_TOK, 1), jnp.float32),
            pltpu.VMEM((CHUNK, D_H), jnp.bfloat16),
            pltpu.VMEM((N_HOPS, CHUNK, D_H), jnp.bfloat16),
            pltpu.SemaphoreType.DMA((N_HOPS,)),
            pltpu.SemaphoreType.DMA((N_HOPS,)),
        ],
        compiler_params=pltpu.CompilerParams(collective_id=0),
    )(x, router_W, route_idx,
      expert_W.astype(jnp.bfloat16), shared_W.astype(jnp.bfloat16))


# device time: 113559 ns/iter; 1.0914x vs baseline; 1.0914x over previous
import jax
import jax.numpy as jnp
from jax import lax
from jax.experimental import pallas as pl
from jax.experimental.pallas import tpu as pltpu

N_DEV = 4
N_TOK = 2048
D_IN = 512
D_H = 1024
N_EXP = 32
E_LOC = N_EXP // N_DEV
CHUNK = N_TOK // N_DEV
N_HOPS = 2 * (N_DEV - 1)


def kernel(x, router_W, route_idx, expert_W, shared_W):
    def body(x_ref, rw_ref, ri_ref, ew_ref, sw_ref, out_ref,
             pt_scr, sbuf, comm, send_sems, recv_sems):
        my = lax.axis_index("i")
        left = lax.rem(my + N_DEV - 1, N_DEV)
        right = lax.rem(my + 1, N_DEV)

        barrier = pltpu.get_barrier_semaphore()
        for nbr in (left, right):
            pl.semaphore_signal(barrier, inc=1, device_id=(nbr,),
                                device_id_type=pl.DeviceIdType.MESH)
        pl.semaphore_wait(barrier, 2)

        scores = jnp.dot(x_ref[:, :], rw_ref[:, :],
                         preferred_element_type=jnp.float32)
        s_max = jnp.max(scores, axis=-1, keepdims=True)
        p = jnp.exp(scores - s_max)
        probs = p / jnp.sum(p, axis=-1, keepdims=True)
        ri = ri_ref[:, :]
        eidx = lax.broadcasted_iota(jnp.int32, (N_TOK, N_EXP), 1)
        pt_scr[:, :] = jnp.sum(jnp.where(eidx == ri, probs, 0.0),
                               axis=-1, keepdims=True)

        e_base = my * E_LOC

        def partial_chunk(c):
            r0 = c * CHUNK
            xb = x_ref[pl.ds(r0, CHUNK), :].astype(jnp.bfloat16)
            ri_c = ri_ref[pl.ds(r0, CHUNK), :]
            pt_c = pt_scr[pl.ds(r0, CHUNK), :]
            acc = jnp.zeros((CHUNK, D_H), jnp.float32)
            for le in range(E_LOC):
                gate = jnp.where(ri_c == e_base + le, pt_c, 0.0)
                acc = acc + jnp.dot(xb * gate.astype(jnp.bfloat16), ew_ref[le],
                                    preferred_element_type=jnp.float32)
            return acc

        def rdma(src, k):
            return pltpu.make_async_remote_copy(
                src_ref=src, dst_ref=comm.at[k],
                send_sem=send_sems.at[k], recv_sem=recv_sems.at[k],
                device_id=(right,), device_id_type=pl.DeviceIdType.MESH)

        ops = []

        for s in range(N_DEV - 1):
            pc = partial_chunk(lax.rem(my - s + N_DEV, N_DEV))
            if s == 0:
                sbuf[0, :, :] = pc.astype(jnp.bfloat16)
            else:
                ops[s - 1].wait_recv()
                sbuf[s, :, :] = (comm[s - 1].astype(jnp.float32)
                                 + pc).astype(jnp.bfloat16)
            ops.append(rdma(sbuf.at[s], s))
            ops[s].start()

        cmine = lax.rem(my + 1, N_DEV)
        pc = partial_chunk(cmine)
        xc = x_ref[pl.ds(cmine * CHUNK, CHUNK), :].astype(jnp.bfloat16)
        shared_c = jnp.dot(xc, sw_ref[:, :], preferred_element_type=jnp.float32)
        ops[N_DEV - 2].wait_recv()
        final = comm[N_DEV - 2].astype(jnp.float32) + pc + shared_c
        sbuf[N_DEV - 1, :, :] = final.astype(jnp.bfloat16)
        ops.append(rdma(sbuf.at[N_DEV - 1], N_DEV - 1))
        ops[N_DEV - 1].start()
        out_ref[pl.ds(cmine * CHUNK, CHUNK), :] = final

        for t in range(1, N_DEV - 1):
            k = (N_DEV - 1) + t
            ops[k - 1].wait_recv()
            ops.append(rdma(comm.at[k - 1], k))
            ops[k].start()
            corig = lax.rem(my - t + 1 + N_DEV, N_DEV)
            out_ref[pl.ds(corig * CHUNK, CHUNK), :] = \
                comm[k - 1].astype(jnp.float32)
        ops[N_HOPS - 1].wait_recv()
        corig = lax.rem(my - (N_DEV - 2) + N_DEV, N_DEV)
        out_ref[pl.ds(corig * CHUNK, CHUNK), :] = \
            comm[N_HOPS - 1].astype(jnp.float32)

        for op in ops:
            op.wait_send()

    return pl.pallas_call(
        body,
        out_shape=jax.ShapeDtypeStruct((N_TOK, D_H), jnp.float32),
        in_specs=[pl.BlockSpec(memory_space=pltpu.VMEM)] * 5,
        out_specs=pl.BlockSpec(memory_space=pltpu.VMEM),
        scratch_shapes=[
            pltpu.VMEM((N_TOK, 1), jnp.float32),
            pltpu.VMEM((N_DEV, CHUNK, D_H), jnp.bfloat16),
            pltpu.VMEM((N_HOPS, CHUNK, D_H), jnp.bfloat16),
            pltpu.SemaphoreType.DMA((N_HOPS,)),
            pltpu.SemaphoreType.DMA((N_HOPS,)),
        ],
        compiler_params=pltpu.CompilerParams(collective_id=0),
    )(x, router_W, route_idx,
      expert_W.astype(jnp.bfloat16), shared_W.astype(jnp.bfloat16))


# device time: 86850 ns/iter; 1.4270x vs baseline; 1.3075x over previous
import jax
import jax.numpy as jnp
from jax import lax
from jax.experimental import pallas as pl
from jax.experimental.pallas import tpu as pltpu

N_DEV = 4
N_TOK = 2048
D_IN = 512
D_H = 1024
HALF = D_H // 2
N_EXP = 32
E_LOC = N_EXP // N_DEV
CHUNK = N_TOK // N_DEV
N_HOPS = 2 * (N_DEV - 1)


def kernel(x, router_W, route_idx, expert_W, shared_W):
    def body(x_ref, rw_ref, ri_ref, ew_ref, sw_ref, out_ref,
             pt_scr, sbL, sbR, cmL, cmR,
             send_semL, recv_semL, send_semR, recv_semR):
        my = lax.axis_index("i")
        left = lax.rem(my + N_DEV - 1, N_DEV)
        right = lax.rem(my + 1, N_DEV)

        barrier = pltpu.get_barrier_semaphore()
        for nbr in (left, right):
            pl.semaphore_signal(barrier, inc=1, device_id=(nbr,),
                                device_id_type=pl.DeviceIdType.MESH)
        pl.semaphore_wait(barrier, 2)

        scores = jnp.dot(x_ref[:, :], rw_ref[:, :],
                         preferred_element_type=jnp.float32)
        s_max = jnp.max(scores, axis=-1, keepdims=True)
        p = jnp.exp(scores - s_max)
        probs = p / jnp.sum(p, axis=-1, keepdims=True)
        ri = ri_ref[:, :]
        eidx = lax.broadcasted_iota(jnp.int32, (N_TOK, N_EXP), 1)
        pt_scr[:, :] = jnp.sum(jnp.where(eidx == ri, probs, 0.0),
                               axis=-1, keepdims=True)

        e_base = my * E_LOC

        def partial_half(c, col0):
            r0 = c * CHUNK
            xb = x_ref[pl.ds(r0, CHUNK), :].astype(jnp.bfloat16)
            ri_c = ri_ref[pl.ds(r0, CHUNK), :]
            pt_c = pt_scr[pl.ds(r0, CHUNK), :]
            acc = jnp.zeros((CHUNK, HALF), jnp.float32)
            for le in range(E_LOC):
                gate = jnp.where(ri_c == e_base + le, pt_c, 0.0)
                acc = acc + jnp.dot(xb * gate.astype(jnp.bfloat16),
                                    ew_ref[le, :, col0:col0 + HALF],
                                    preferred_element_type=jnp.float32)
            return acc

        def rdmaL(src, k):
            return pltpu.make_async_remote_copy(
                src_ref=src, dst_ref=cmL.at[k],
                send_sem=send_semL.at[k], recv_sem=recv_semL.at[k],
                device_id=(right,), device_id_type=pl.DeviceIdType.MESH)

        def rdmaR(src, k):
            return pltpu.make_async_remote_copy(
                src_ref=src, dst_ref=cmR.at[k],
                send_sem=send_semR.at[k], recv_sem=recv_semR.at[k],
                device_id=(left,), device_id_type=pl.DeviceIdType.MESH)

        opsL, opsR = [], []

        for s in range(N_DEV - 1):
            pcL = partial_half(lax.rem(my - s + N_DEV, N_DEV), 0)
            if s == 0:
                sbL[0, :, :] = pcL.astype(jnp.bfloat16)
            else:
                opsL[s - 1].wait_recv()
                sbL[s, :, :] = (cmL[s - 1].astype(jnp.float32)
                                + pcL).astype(jnp.bfloat16)
            opsL.append(rdmaL(sbL.at[s], s))
            opsL[s].start()

            pcR = partial_half(lax.rem(my + s, N_DEV), HALF)
            if s == 0:
                sbR[0, :, :] = pcR.astype(jnp.bfloat16)
            else:
                opsR[s - 1].wait_recv()
                sbR[s, :, :] = (cmR[s - 1].astype(jnp.float32)
                                + pcR).astype(jnp.bfloat16)
            opsR.append(rdmaR(sbR.at[s], s))
            opsR[s].start()

        for ring in ("L", "R"):
            if ring == "L":
                cmine, col0 = lax.rem(my + 1, N_DEV), 0
                ops, sb, cm, rd = opsL, sbL, cmL, rdmaL
            else:
                cmine, col0 = lax.rem(my + N_DEV - 1, N_DEV), HALF
                ops, sb, cm, rd = opsR, sbR, cmR, rdmaR
            pc = partial_half(cmine, col0)
            xc = x_ref[pl.ds(cmine * CHUNK, CHUNK), :].astype(jnp.bfloat16)
            shared_c = jnp.dot(xc, sw_ref[:, col0:col0 + HALF],
                               preferred_element_type=jnp.float32)
            ops[N_DEV - 2].wait_recv()
            final = cm[N_DEV - 2].astype(jnp.float32) + pc + shared_c
            sb[N_DEV - 1, :, :] = final.astype(jnp.bfloat16)
            ops.append(rd(sb.at[N_DEV - 1], N_DEV - 1))
            ops[N_DEV - 1].start()
            out_ref[pl.ds(cmine * CHUNK, CHUNK), col0:col0 + HALF] = final

        for j in range(1, N_DEV - 1):
            k = (N_DEV - 1) + j
            opsL[k - 1].wait_recv()
            opsL.append(rdmaL(cmL.at[k - 1], k))
            opsL[k].start()
            cL = lax.rem(my - (j - 1) + N_DEV, N_DEV)
            out_ref[pl.ds(cL * CHUNK, CHUNK), 0:HALF] = \
                cmL[k - 1].astype(jnp.float32)

            opsR[k - 1].wait_recv()
            opsR.append(rdmaR(cmR.at[k - 1], k))
            opsR[k].start()
            cR = lax.rem(my + (j - 1), N_DEV)
            out_ref[pl.ds(cR * CHUNK, CHUNK), HALF:D_H] = \
                cmR[k - 1].astype(jnp.float32)

        opsL[N_HOPS - 1].wait_recv()
        cL = lax.rem(my - (N_DEV - 2) + N_DEV, N_DEV)
        out_ref[pl.ds(cL * CHUNK, CHUNK), 0:HALF] = \
            cmL[N_HOPS - 1].astype(jnp.float32)
        opsR[N_HOPS - 1].wait_recv()
        cR = lax.rem(my + N_DEV - 2, N_DEV)
        out_ref[pl.ds(cR * CHUNK, CHUNK), HALF:D_H] = \
            cmR[N_HOPS - 1].astype(jnp.float32)

        for op in opsL + opsR:
            op.wait_send()

    return pl.pallas_call(
        body,
        out_shape=jax.ShapeDtypeStruct((N_TOK, D_H), jnp.float32),
        in_specs=[pl.BlockSpec(memory_space=pltpu.VMEM)] * 5,
        out_specs=pl.BlockSpec(memory_space=pltpu.VMEM),
        scratch_shapes=[
            pltpu.VMEM((N_TOK, 1), jnp.float32),
            pltpu.VMEM((N_DEV, CHUNK, HALF), jnp.bfloat16),
            pltpu.VMEM((N_DEV, CHUNK, HALF), jnp.bfloat16),
            pltpu.VMEM((N_HOPS, CHUNK, HALF), jnp.bfloat16),
            pltpu.VMEM((N_HOPS, CHUNK, HALF), jnp.bfloat16),
            pltpu.SemaphoreType.DMA((N_HOPS,)),
            pltpu.SemaphoreType.DMA((N_HOPS,)),
            pltpu.SemaphoreType.DMA((N_HOPS,)),
            pltpu.SemaphoreType.DMA((N_HOPS,)),
        ],
        compiler_params=pltpu.CompilerParams(collective_id=0),
    )(x, router_W, route_idx,
      expert_W.astype(jnp.bfloat16), shared_W.astype(jnp.bfloat16))


# device time: 80914 ns/iter; 1.5317x vs baseline; 1.0734x over previous
import jax
import jax.numpy as jnp
from jax import lax
from jax.experimental import pallas as pl
from jax.experimental.pallas import tpu as pltpu

N_DEV = 4
N_TOK = 2048
D_IN = 512
D_H = 1024
HALF = D_H // 2
N_EXP = 32
E_LOC = N_EXP // N_DEV
CHUNK = N_TOK // N_DEV
N_HOPS = 2 * (N_DEV - 1)


def kernel(x, router_W, route_idx, expert_W, shared_W):
    def body(x_ref, rw_ref, ri_ref, ew_ref, sw_ref, out_ref,
             pt_scr, xb_scr, sbL, sbR, cmL, cmR,
             send_semL, recv_semL, send_semR, recv_semR):
        my = lax.axis_index("i")
        left = lax.rem(my + N_DEV - 1, N_DEV)
        right = lax.rem(my + 1, N_DEV)

        barrier = pltpu.get_barrier_semaphore()
        for nbr in (left, right):
            pl.semaphore_signal(barrier, inc=1, device_id=(nbr,),
                                device_id_type=pl.DeviceIdType.MESH)
        pl.semaphore_wait(barrier, 2)

        xb_scr[:, :] = x_ref[:, :].astype(jnp.bfloat16)

        scores = jnp.dot(xb_scr[:, :], rw_ref[:, :].astype(jnp.bfloat16),
                         preferred_element_type=jnp.float32)
        s_max = jnp.max(scores, axis=-1, keepdims=True)
        p = jnp.exp(scores - s_max)
        probs = p / jnp.sum(p, axis=-1, keepdims=True)
        ri = ri_ref[:, :]
        eidx = lax.broadcasted_iota(jnp.int32, (N_TOK, N_EXP), 1)
        pt_scr[:, :] = jnp.sum(jnp.where(eidx == ri, probs, 0.0),
                               axis=-1, keepdims=True)

        e_base = my * E_LOC

        def partial_half(c, col0):
            r0 = c * CHUNK
            xb = xb_scr[pl.ds(r0, CHUNK), :]
            ri_c = ri_ref[pl.ds(r0, CHUNK), :]
            pt_c = pt_scr[pl.ds(r0, CHUNK), :]
            acc = jnp.zeros((CHUNK, HALF), jnp.float32)
            for le in range(E_LOC):
                gate = jnp.where(ri_c == e_base + le, pt_c, 0.0)
                acc = acc + jnp.dot(xb * gate.astype(jnp.bfloat16),
                                    ew_ref[le, :, col0:col0 + HALF],
                                    preferred_element_type=jnp.float32)
            return acc

        def rdmaL(src, dst, k):
            return pltpu.make_async_remote_copy(
                src_ref=src, dst_ref=dst,
                send_sem=send_semL.at[k], recv_sem=recv_semL.at[k],
                device_id=(right,), device_id_type=pl.DeviceIdType.MESH)

        def rdmaR(src, dst, k):
            return pltpu.make_async_remote_copy(
                src_ref=src, dst_ref=dst,
                send_sem=send_semR.at[k], recv_sem=recv_semR.at[k],
                device_id=(left,), device_id_type=pl.DeviceIdType.MESH)

        def out_at(c, col0):
            cs = slice(0, HALF) if col0 == 0 else slice(HALF, D_H)
            return out_ref.at[pl.ds(c * CHUNK, CHUNK), cs]

        opsL, opsR = [], []

        for s in range(N_DEV - 1):
            pcL = partial_half(lax.rem(my - s + N_DEV, N_DEV), 0)
            if s == 0:
                sbL[0, :, :] = pcL.astype(jnp.bfloat16)
            else:
                opsL[s - 1].wait_recv()
                sbL[s, :, :] = (cmL[s - 1].astype(jnp.float32)
                                + pcL).astype(jnp.bfloat16)
            opsL.append(rdmaL(sbL.at[s], cmL.at[s], s))
            opsL[s].start()

            pcR = partial_half(lax.rem(my + s, N_DEV), HALF)
            if s == 0:
                sbR[0, :, :] = pcR.astype(jnp.bfloat16)
            else:
                opsR[s - 1].wait_recv()
                sbR[s, :, :] = (cmR[s - 1].astype(jnp.float32)
                                + pcR).astype(jnp.bfloat16)
            opsR.append(rdmaR(sbR.at[s], cmR.at[s], s))
            opsR[s].start()

        for ring in ("L", "R"):
            if ring == "L":
                cmine, col0 = lax.rem(my + 1, N_DEV), 0
                ops, cm, rd = opsL, cmL, rdmaL
            else:
                cmine, col0 = lax.rem(my + N_DEV - 1, N_DEV), HALF
                ops, cm, rd = opsR, cmR, rdmaR
            pc = partial_half(cmine, col0)
            xc = xb_scr[pl.ds(cmine * CHUNK, CHUNK), :]
            shared_c = jnp.dot(xc, sw_ref[:, col0:col0 + HALF],
                               preferred_element_type=jnp.float32)
            ops[N_DEV - 2].wait_recv()
            final = cm[N_DEV - 2].astype(jnp.float32) + pc + shared_c
            cs = slice(0, HALF) if col0 == 0 else slice(HALF, D_H)
            out_ref[pl.ds(cmine * CHUNK, CHUNK), cs] = final.astype(jnp.bfloat16)
            ops.append(rd(out_at(cmine, col0), out_at(cmine, col0),
                          N_DEV - 1))
            ops[N_DEV - 1].start()

        for a in range(1, N_DEV - 1):
            k = (N_DEV - 1) + a
            opsL[k - 1].wait_recv()
            csend = lax.rem(my + 1 - a + N_DEV, N_DEV)
            opsL.append(rdmaL(out_at(csend, 0), out_at(csend, 0), k))
            opsL[k].start()

            opsR[k - 1].wait_recv()
            csend = lax.rem(my - 1 + a + N_DEV, N_DEV)
            opsR.append(rdmaR(out_at(csend, HALF), out_at(csend, HALF), k))
            opsR[k].start()

        opsL[N_HOPS - 1].wait_recv()
        opsR[N_HOPS - 1].wait_recv()
        for op in opsL + opsR:
            op.wait_send()

    return pl.pallas_call(
        body,
        out_shape=jax.ShapeDtypeStruct((N_TOK, D_H), jnp.bfloat16),
        in_specs=[pl.BlockSpec(memory_space=pltpu.VMEM)] * 5,
        out_specs=pl.BlockSpec(memory_space=pltpu.VMEM),
        scratch_shapes=[
            pltpu.VMEM((N_TOK, 1), jnp.float32),
            pltpu.VMEM((N_TOK, D_IN), jnp.bfloat16),
            pltpu.VMEM((N_DEV - 1, CHUNK, HALF), jnp.bfloat16),
            pltpu.VMEM((N_DEV - 1, CHUNK, HALF), jnp.bfloat16),
            pltpu.VMEM((N_DEV - 1, CHUNK, HALF), jnp.bfloat16),
            pltpu.VMEM((N_DEV - 1, CHUNK, HALF), jnp.bfloat16),
            pltpu.SemaphoreType.DMA((N_HOPS,)),
            pltpu.SemaphoreType.DMA((N_HOPS,)),
            pltpu.SemaphoreType.DMA((N_HOPS,)),
            pltpu.SemaphoreType.DMA((N_HOPS,)),
        ],
        compiler_params=pltpu.CompilerParams(collective_id=0),
    )(x, router_W, route_idx,
      expert_W.astype(jnp.bfloat16), shared_W.astype(jnp.bfloat16))


# device time: 75992 ns/iter; 1.6309x vs baseline; 1.0648x over previous
import jax
import jax.numpy as jnp
from jax import lax
from jax.experimental import pallas as pl
from jax.experimental.pallas import tpu as pltpu

N_DEV = 4
N_TOK = 2048
D_IN = 512
D_H = 1024
HALF = D_H // 2
N_EXP = 32
E_LOC = N_EXP // N_DEV
CHUNK = N_TOK // N_DEV
N_HOPS = 2 * (N_DEV - 1)


def kernel(x, router_W, route_idx, expert_W, shared_W):
    def body(x_ref, rw_ref, ri_ref, ew_ref, sw_ref, out_ref,
             pt_scr, xb_scr, sbL, sbR, cmL, cmR,
             send_semL, recv_semL, send_semR, recv_semR):
        my = lax.axis_index("i")
        left = lax.rem(my + N_DEV - 1, N_DEV)
        right = lax.rem(my + 1, N_DEV)

        barrier = pltpu.get_barrier_semaphore()
        for nbr in (left, right):
            pl.semaphore_signal(barrier, inc=1, device_id=(nbr,),
                                device_id_type=pl.DeviceIdType.MESH)
        pl.semaphore_wait(barrier, 2)

        xb_scr[:, :] = x_ref[:, :].astype(jnp.bfloat16)

        scores = jnp.dot(xb_scr[:, :], rw_ref[:, :].astype(jnp.bfloat16),
                         preferred_element_type=jnp.float32)
        s_max = jnp.max(scores, axis=-1, keepdims=True)
        p = jnp.exp(scores - s_max)
        probs = p / jnp.sum(p, axis=-1, keepdims=True)
        ri = ri_ref[:, :]
        eidx = lax.broadcasted_iota(jnp.int32, (N_TOK, N_EXP), 1)
        pt_scr[:, :] = jnp.sum(jnp.where(eidx == ri, probs, 0.0),
                               axis=-1, keepdims=True)

        e_base = my * E_LOC
        SUB = CHUNK // 2

        def partial_half(c, col0, sub):
            r0 = c * CHUNK + sub * SUB
            xb = xb_scr[pl.ds(r0, SUB), :]
            ri_c = ri_ref[pl.ds(r0, SUB), :]
            pt_c = pt_scr[pl.ds(r0, SUB), :]
            acc = jnp.zeros((SUB, HALF), jnp.float32)
            for le in range(E_LOC):
                gate = jnp.where(ri_c == e_base + le, pt_c, 0.0)
                acc = acc + jnp.dot(
                    xb * gate.astype(jnp.bfloat16),
                    ew_ref[le, :, col0:col0 + HALF].astype(jnp.bfloat16),
                    preferred_element_type=jnp.float32)
            return acc

        def rdmaL(src, dst, k):
            return pltpu.make_async_remote_copy(
                src_ref=src, dst_ref=dst,
                send_sem=send_semL.at[k], recv_sem=recv_semL.at[k],
                device_id=(right,), device_id_type=pl.DeviceIdType.MESH)

        def rdmaR(src, dst, k):
            return pltpu.make_async_remote_copy(
                src_ref=src, dst_ref=dst,
                send_sem=send_semR.at[k], recv_sem=recv_semR.at[k],
                device_id=(left,), device_id_type=pl.DeviceIdType.MESH)

        opsL, opsR = [], []
        rsub = slice(0, SUB), slice(SUB, CHUNK)

        for s in range(N_DEV - 1):
            cL = lax.rem(my - s + N_DEV, N_DEV)
            cR = lax.rem(my + s, N_DEV)
            for sub in range(2):
                j = (s - 1) * 2 + sub
                k = s * 2 + sub
                pcL = partial_half(cL, 0, sub)
                if s == 0:
                    sbL[0, rsub[sub], :] = pcL.astype(jnp.bfloat16)
                else:
                    opsL[j].wait_recv()
                    sbL[s, rsub[sub], :] = (
                        cmL[s - 1, rsub[sub], :].astype(jnp.float32)
                        + pcL).astype(jnp.bfloat16)
                opsL.append(rdmaL(sbL.at[s, pl.ds(sub * SUB, SUB), :],
                                  cmL.at[s, pl.ds(sub * SUB, SUB), :], k))
                opsL[k].start()

                pcR = partial_half(cR, HALF, sub)
                if s == 0:
                    sbR[0, rsub[sub], :] = pcR.astype(jnp.bfloat16)
                else:
                    opsR[j].wait_recv()
                    sbR[s, rsub[sub], :] = (
                        cmR[s - 1, rsub[sub], :].astype(jnp.float32)
                        + pcR).astype(jnp.bfloat16)
                opsR.append(rdmaR(sbR.at[s, pl.ds(sub * SUB, SUB), :],
                                  cmR.at[s, pl.ds(sub * SUB, SUB), :], k))
                opsR[k].start()

        def out_sub(c, col0, sub):
            cs = slice(0, HALF) if col0 == 0 else slice(HALF, D_H)
            return out_ref.at[pl.ds(c * CHUNK + sub * SUB, SUB), cs]

        agL, agR = [], []
        for sub in range(2):
            for ring in ("L", "R"):
                if ring == "L":
                    cmine, col0 = lax.rem(my + 1, N_DEV), 0
                    ops, cm, rd, ag = opsL, cmL, rdmaL, agL
                else:
                    cmine, col0 = lax.rem(my + N_DEV - 1, N_DEV), HALF
                    ops, cm, rd, ag = opsR, cmR, rdmaR, agR
                pc = partial_half(cmine, col0, sub)
                xc = xb_scr[pl.ds(cmine * CHUNK + sub * SUB, SUB), :]
                shared_c = jnp.dot(
                    xc, sw_ref[:, col0:col0 + HALF].astype(jnp.bfloat16),
                    preferred_element_type=jnp.float32)
                ops[(N_DEV - 2) * 2 + sub].wait_recv()
                final = (cm[N_DEV - 2, rsub[sub], :].astype(jnp.float32)
                         + pc + shared_c)
                cs = slice(0, HALF) if col0 == 0 else slice(HALF, D_H)
                out_ref[pl.ds(cmine * CHUNK + sub * SUB, SUB), cs] = \
                    final.astype(jnp.bfloat16)
                ag.append(rd(out_sub(cmine, col0, sub),
                             out_sub(cmine, col0, sub), 6 + sub))
                ag[sub].start()

        for a in range(1, N_DEV - 1):
            cL = lax.rem(my + 1 - a + N_DEV, N_DEV)
            cR = lax.rem(my - 1 + a + N_DEV, N_DEV)
            for sub in range(2):
                j = (a - 1) * 2 + sub
                k = 6 + a * 2 + sub
                agL[j].wait_recv()
                agL.append(rdmaL(out_sub(cL, 0, sub), out_sub(cL, 0, sub), k))
                agL[a * 2 + sub].start()
                agR[j].wait_recv()
                agR.append(rdmaR(out_sub(cR, HALF, sub),
                                 out_sub(cR, HALF, sub), k))
                agR[a * 2 + sub].start()

        for j in ((N_DEV - 2) * 2, (N_DEV - 2) * 2 + 1):
            agL[j].wait_recv()
            agR[j].wait_recv()
        for op in opsL + opsR + agL + agR:
            op.wait_send()

    return pl.pallas_call(
        body,
        out_shape=jax.ShapeDtypeStruct((N_TOK, D_H), jnp.bfloat16),
        in_specs=[pl.BlockSpec(memory_space=pltpu.VMEM)] * 5,
        out_specs=pl.BlockSpec(memory_space=pltpu.VMEM),
        scratch_shapes=[
            pltpu.VMEM((N_TOK, 1), jnp.float32),
            pltpu.VMEM((N_TOK, D_IN), jnp.bfloat16),
            pltpu.VMEM((N_DEV - 1, CHUNK, HALF), jnp.bfloat16),
            pltpu.VMEM((N_DEV - 1, CHUNK, HALF), jnp.bfloat16),
            pltpu.VMEM((N_DEV - 1, CHUNK, HALF), jnp.bfloat16),
            pltpu.VMEM((N_DEV - 1, CHUNK, HALF), jnp.bfloat16),
            pltpu.SemaphoreType.DMA((12,)),
            pltpu.SemaphoreType.DMA((12,)),
            pltpu.SemaphoreType.DMA((12,)),
            pltpu.SemaphoreType.DMA((12,)),
        ],
        compiler_params=pltpu.CompilerParams(collective_id=0),
    )(x, router_W, route_idx, expert_W, shared_W)
